# baseline (speedup 1.0000x reference)
import functools

import jax
import jax.numpy as jnp
from jax import lax
from jax.experimental import pallas as pl
from jax.experimental.pallas import tpu as pltpu
from jax.experimental.pallas import tpu_sc as plsc

BATCH = 16384
EMB_DIM = 64
LANES = 16

_NUM_CORES = 2
_NUM_SUBCORES = 16
_NUM_WORKERS = _NUM_CORES * _NUM_SUBCORES
_BPW = BATCH // _NUM_WORKERS
_CHUNK = 128
_NCHUNKS = _BPW // _CHUNK
_CGROUPS = _CHUNK // LANES


def _make_kernel():
    mesh = plsc.VectorSubcoreMesh(core_axis_name="c", subcore_axis_name="s")

    @functools.partial(
        pl.kernel,
        mesh=mesh,
        out_type=jax.ShapeDtypeStruct((BATCH,), jnp.float32),
        scratch_types=[
            pltpu.VMEM((_BPW,), jnp.int32),
            pltpu.VMEM((_BPW,), jnp.int32),
            pltpu.VMEM((2, _CHUNK, EMB_DIM), jnp.float32),
            pltpu.VMEM((2, _CHUNK, EMB_DIM), jnp.float32),
            pltpu.VMEM((_BPW,), jnp.float32),
            pltpu.SemaphoreType.DMA,
            pltpu.SemaphoreType.DMA,
            pltpu.SemaphoreType.DMA,
            pltpu.SemaphoreType.DMA,
        ],
    )
    def two_tower(uid_hbm, iid_hbm, uemb_hbm, iemb_hbm, out_hbm,
                  uid_v, iid_v, ubuf, ibuf, scores, su0, su1, si0, si1):
        wid = lax.axis_index("s") * _NUM_CORES + lax.axis_index("c")
        base = wid * _BPW

        pltpu.sync_copy(uid_hbm.at[pl.ds(base, _BPW)], uid_v)
        pltpu.sync_copy(iid_hbm.at[pl.ds(base, _BPW)], iid_v)

        sus = (su0, su1)
        sis = (si0, si1)

        def issue_chunk(c, slot):
            def igroup(g, _):
                off = c * _CHUNK + g * LANES
                idu = uid_v[pl.ds(off, LANES)]
                idi = iid_v[pl.ds(off, LANES)]
                for k in range(LANES):
                    dst = g * LANES + k
                    pltpu.async_copy(uemb_hbm.at[idu[k]],
                                     ubuf.at[slot, dst], sus[slot])
                    pltpu.async_copy(iemb_hbm.at[idi[k]],
                                     ibuf.at[slot, dst], sis[slot])
                return _

            lax.fori_loop(0, _CGROUPS, igroup, 0)

        def drain_chunk(slot):
            def dgroup(j, _):
                pltpu.make_async_copy(uemb_hbm.at[0], ubuf.at[slot, 0],
                                      sus[slot]).wait()
                pltpu.make_async_copy(iemb_hbm.at[0], ibuf.at[slot, 0],
                                      sis[slot]).wait()
                return _

            lax.fori_loop(0, _CHUNK, dgroup, 0)

        lane_iota = lax.iota(jnp.int32, LANES)
        perms = [lane_iota ^ bit for bit in (1, 2, 4, 8)]
        dnums = lax.GatherDimensionNumbers(
            offset_dims=(), collapsed_slice_dims=(0,), start_index_map=(0,))

        def lane_perm(v, perm):
            return lax.gather(
                v, perm[:, None], dnums, (1,),
                indices_are_sorted=False, unique_indices=False,
                mode=lax.GatherScatterMode.PROMISE_IN_BOUNDS)

        def compute_chunk(c, slot):
            def group(g, _):
                vecs = []
                for k in range(LANES):
                    j = g * LANES + k
                    a0 = (ubuf[slot, j, pl.ds(0, 16)]
                          * ibuf[slot, j, pl.ds(0, 16)])
                    a1 = (ubuf[slot, j, pl.ds(16, 16)]
                          * ibuf[slot, j, pl.ds(16, 16)])
                    a2 = (ubuf[slot, j, pl.ds(32, 16)]
                          * ibuf[slot, j, pl.ds(32, 16)])
                    a3 = (ubuf[slot, j, pl.ds(48, 16)]
                          * ibuf[slot, j, pl.ds(48, 16)])
                    vecs.append((a0 + a1) + (a2 + a3))
                for bit, perm in zip((1, 2, 4, 8), perms):
                    keep = (lane_iota & bit) == 0
                    nxt = []
                    for p in range(0, len(vecs), 2):
                        a, b = vecs[p], vecs[p + 1]
                        ap = a + lane_perm(a, perm)
                        bp = b + lane_perm(b, perm)
                        nxt.append(jnp.where(keep, ap, bp))
                    vecs = nxt
                vec = vecs[0]
                vec = 1.0 / (1.0 + jnp.exp(-vec))
                scores[pl.ds(c * _CHUNK + g * LANES, LANES)] = vec
                return _

            lax.fori_loop(0, _CGROUPS, group, 0)

        issue_chunk(0, 0)
        for c in range(_NCHUNKS):
            slot = c & 1
            if c + 1 < _NCHUNKS:
                issue_chunk(c + 1, (c + 1) & 1)
            drain_chunk(slot)
            compute_chunk(c, slot)

        pltpu.sync_copy(scores, out_hbm.at[pl.ds(base, _BPW)])

    return two_tower


_TWO_TOWER = _make_kernel()


def kernel(user_ids, item_ids, user_emb, item_emb):
    return _TWO_TOWER(user_ids.astype(jnp.int32), item_ids.astype(jnp.int32),
                      user_emb, item_emb)
